# baseline (speedup 1.0000x reference)
import jax
import jax.numpy as jnp
from jax.experimental import pallas as pl
from jax.experimental.pallas import tpu as pltpu

B, T, N, H = 4, 16, 10000, 64
S = 5
NS = N // S


def _body(sp_ref, stack_ref, hid_ref, hint_ref, spv_ref, out_ref, ptr_ref):
    b = pl.program_id(0)
    s = pl.program_id(1)
    t = pl.program_id(2)
    tgt = sp_ref[b] + 1

    @pl.when(t == tgt)
    def _():
        out_ref[...] = hid_ref[...]

    @pl.when(t != tgt)
    def _():
        out_ref[...] = stack_ref[...]

    @pl.when((b == 0) & (s == 0) & (t == 0))
    def _():
        h = hint_ref[...]
        a0 = h[:, :, 0]
        a1 = h[:, :, 1]
        a2 = h[:, :, 2]
        ops = jnp.where(a0 >= a1,
                        jnp.where(a0 >= a2, 0, 2),
                        jnp.where(a1 >= a2, 1, 2)).astype(jnp.int32)
        ptr_ref[...] = jnp.maximum(spv_ref[...] + ops - 1, 0)


def kernel(stack, stack_pointers, hint_preds, hiddens, graph_fts):
    del graph_fts
    sp_flat = jnp.reshape(stack_pointers, (B,))

    def stack_idx(b, s, t, sp):
        tt = jnp.where(t == sp[b] + 1, t - 1, t)
        return (b, tt, s, 0)

    grid_spec = pltpu.PrefetchScalarGridSpec(
        num_scalar_prefetch=1,
        grid=(B, S, T),
        in_specs=[
            pl.BlockSpec((1, 1, NS, H), stack_idx),
            pl.BlockSpec((1, 1, NS, H), lambda b, s, t, sp: (0, b, s, 0)),
            pl.BlockSpec((1, B, 3), lambda b, s, t, sp: (0, 0, 0)),
            pl.BlockSpec((1, B), lambda b, s, t, sp: (0, 0)),
        ],
        out_specs=[
            pl.BlockSpec((1, 1, NS, H), lambda b, s, t, sp: (b, t, s, 0)),
            pl.BlockSpec((1, B), lambda b, s, t, sp: (0, 0)),
        ],
    )

    new_stack, new_ptrs = pl.pallas_call(
        _body,
        grid_spec=grid_spec,
        out_shape=[
            jax.ShapeDtypeStruct((B, T, N, H), jnp.float32),
            jax.ShapeDtypeStruct((1, B), jnp.int32),
        ],
    )(sp_flat, stack, hiddens, hint_preds, stack_pointers)
    return (new_stack, new_ptrs)
